# baseline (speedup 1.0000x reference)
import functools

import jax
import jax.numpy as jnp
from jax import lax
from jax.experimental import pallas as pl
from jax.experimental.pallas import tpu as pltpu
from jax.experimental.pallas import tpu_sc as plsc

TQ = 512
QT = 512
K = 16
NW = 32
SC_CHUNK = 128


def _knn_body(aT_ref, s_ref, idx_ref, *, M):
    b = pl.program_id(0)
    q = aT_ref[0]
    s = s_ref[0]
    inner = lax.dot_general(q.astype(jnp.bfloat16), s.astype(jnp.bfloat16),
                            (((1,), (0,)), ((), ())),
                            preferred_element_type=jnp.float32)
    qq = jnp.sum(q * q, axis=1, keepdims=True)
    ss = jnp.sum(s * s, axis=0, keepdims=True)
    p = (2.0 * inner - qq) - ss

    iota = lax.broadcasted_iota(jnp.int32, (TQ, M), 1).astype(jnp.float32)
    bigf = jnp.float32(M)
    cols = []
    for t in range(K):
        m = jnp.max(p, axis=1, keepdims=True)
        cand = jnp.where(p == m, iota, bigf)
        sel = jnp.min(cand, axis=1, keepdims=True)
        cols.append(sel)
        if t < K - 1:
            p = jnp.where(iota == sel, -jnp.inf, p)
    idx = jnp.concatenate(cols, axis=1).astype(jnp.int32)
    idx_ref[0] = idx + b * jnp.int32(M)


def _knn_call(aT, s):
    B, N, C = aT.shape
    M = s.shape[2]
    nt = N // TQ
    return pl.pallas_call(
        functools.partial(_knn_body, M=M),
        grid=(B, nt),
        in_specs=[
            pl.BlockSpec((1, TQ, C), lambda b, t: (b, t, 0)),
            pl.BlockSpec((1, C, M), lambda b, t: (b, 0, 0)),
        ],
        out_specs=pl.BlockSpec((1, TQ, K), lambda b, t: (b, t, 0)),
        out_shape=jax.ShapeDtypeStruct((B, N, K), jnp.int32),
    )(aT, s)


def _srl(x, n):
    return lax.shift_right_logical(x, jnp.full(x.shape, n, x.dtype))


NBUF = 8


def _bf16_round(v):
    u = plsc.bitcast(v, jnp.int32)
    u = u + jnp.int32(0x7FFF) + (_srl(u, 16) & jnp.int32(1))
    u = u & jnp.int32(-65536)
    return plsc.bitcast(u, jnp.float32)


def _sc_diff_body(tab_hbm, idx_hbm, dout_hbm,
                  own_v, idx_v, rows_v, dbuf_v, gsem, osem, *, tw, c_real, qw):
    wid = lax.axis_index("s") * 2 + lax.axis_index("c")
    nchunks = (qw * K) // SC_CHUNK
    qpc = SC_CHUNK // K
    nv = tw // 16
    pltpu.sync_copy(tab_hbm.at[pl.ds(wid * qw, qw)], own_v)
    pltpu.sync_copy(idx_hbm.at[pl.ds(wid * nchunks, nchunks)], idx_v)

    for bi in range(NBUF):
        pltpu.async_copy(tab_hbm.at[idx_v.at[bi]], rows_v.at[bi], gsem)

    def group(g, carry):
        for bi in range(NBUF):
            cc = g * NBUF + bi
            pltpu.make_async_copy(tab_hbm.at[pl.ds(0, SC_CHUNK)],
                                  rows_v.at[bi], gsem).wait()
            @pl.when(cc >= NBUF)
            def _drain():
                pltpu.make_async_copy(
                    dbuf_v.at[bi],
                    dout_hbm.at[pl.ds(0, SC_CHUNK)], osem).wait()

            def qloop(q, inner_carry):
                qg = cc * qpc + q
                for v in range(nv):
                    fn = own_v[qg, pl.ds(v * 16, 16)]
                    for i in range(K):
                        r = q * K + i
                        fj = rows_v[bi, r, pl.ds(v * 16, 16)]
                        dbuf_v[bi, r, pl.ds(v * 16, 16)] = _bf16_round(fj - fn)
                return inner_carry

            lax.fori_loop(0, qpc, qloop, 0)
            pltpu.async_copy(
                dbuf_v.at[bi],
                dout_hbm.at[pl.ds(wid * qw * K + cc * SC_CHUNK, SC_CHUNK)],
                osem)
            @pl.when(cc + NBUF < nchunks)
            def _prefetch():
                pltpu.async_copy(tab_hbm.at[idx_v.at[cc + NBUF]],
                                 rows_v.at[bi], gsem)
        return carry

    lax.fori_loop(0, nchunks // NBUF, group, 0)
    for bi in range(NBUF):
        pltpu.make_async_copy(dbuf_v.at[bi], dout_hbm.at[pl.ds(0, SC_CHUNK)],
                              osem).wait()


def _sc_diff(table, idx_flat, c_real):
    rows, tw = table.shape
    nq = idx_flat.shape[0] // K
    qw = nq // NW
    idx2d = idx_flat.reshape(nq * K // SC_CHUNK, SC_CHUNK)
    mesh = plsc.VectorSubcoreMesh(core_axis_name="c", subcore_axis_name="s")
    fn = pl.kernel(
        functools.partial(_sc_diff_body, tw=tw, c_real=c_real, qw=qw),
        mesh=mesh,
        compiler_params=pltpu.CompilerParams(use_tc_tiling_on_sc=False,
                                             needs_layout_passes=False),
        out_type=jax.ShapeDtypeStruct((nq * K, tw), jnp.float32),
        scratch_types=[
            pltpu.VMEM((qw, tw), jnp.float32),
            pltpu.VMEM((qw * K // SC_CHUNK, SC_CHUNK), jnp.int32),
            pltpu.VMEM((NBUF, SC_CHUNK, tw), jnp.float32),
            pltpu.VMEM((NBUF, SC_CHUNK, tw), jnp.float32),
            pltpu.SemaphoreType.DMA,
            pltpu.SemaphoreType.DMA,
        ],
    )
    return fn(table, idx2d)


def _conv_stats_body(d_ref, f_ref, w_ref, smax_ref, acc_ref, *, C):
    d = d_ref[...]
    w = w_ref[...].astype(jnp.bfloat16)
    wl = w[:, :C]
    wr = w[:, C:]
    db = d[:, :C].astype(jnp.bfloat16)
    xd = lax.dot_general(db, wl, (((1,), (1,)), ((), ())),
                         preferred_element_type=jnp.float32)
    f = f_ref[...].astype(jnp.bfloat16)
    hc = lax.dot_general(f, wr, (((1,), (1,)), ((), ())),
                         preferred_element_type=jnp.float32)
    c1 = xd.shape[1]
    x3 = xd.reshape(QT, K, c1)
    smax0 = jnp.max(x3, axis=1)
    ssum0 = jnp.sum(x3, axis=1)
    ssq0 = jnp.sum(x3 * x3, axis=1)
    kf = jnp.float32(K)
    smax_ref[...] = smax0 + hc
    ssum = ssum0 + kf * hc
    ssq = ssq0 + 2.0 * hc * ssum0 + kf * hc * hc
    part = jnp.concatenate([jnp.sum(ssum, axis=0, keepdims=True),
                            jnp.sum(ssq, axis=0, keepdims=True)], axis=0)

    @pl.when(pl.program_id(0) == 0)
    def _init():
        acc_ref[...] = jnp.zeros_like(acc_ref)

    acc_ref[...] += part


def _conv_stats(dpair, feat, w):
    nqk, tw = dpair.shape
    nq = nqk // K
    C = feat.shape[1]
    c1 = w.shape[0]
    return pl.pallas_call(
        functools.partial(_conv_stats_body, C=C),
        grid=(nq // QT,),
        in_specs=[
            pl.BlockSpec((QT * K, tw), lambda r: (r, 0)),
            pl.BlockSpec((QT, C), lambda r: (r, 0)),
            pl.BlockSpec((c1, 2 * C), lambda r: (0, 0)),
        ],
        out_specs=[
            pl.BlockSpec((QT, c1), lambda r: (r, 0)),
            pl.BlockSpec((2, c1), lambda r: (0, 0)),
        ],
        out_shape=[
            jax.ShapeDtypeStruct((nq, c1), jnp.float32),
            jax.ShapeDtypeStruct((2, c1), jnp.float32),
        ],
    )(dpair, feat, w)


def _apply_body(smax_ref, acc_ref, gam_ref, bet_ref, out_ref, *, cnt):
    x = smax_ref[...]
    cntf = jnp.float32(cnt)
    mean = acc_ref[0:1, :] / cntf
    var = acc_ref[1:2, :] / cntf - mean * mean
    rstd = 1.0 / jnp.sqrt(var + 1e-5)
    y = (x - mean) * rstd * gam_ref[...] + bet_ref[...]
    out_ref[...] = jnp.where(y >= 0, y, 0.2 * y)


def _apply_call(smax, acc, gamma, beta):
    rows, c1 = smax.shape
    tr = min(2048, rows)
    return pl.pallas_call(
        functools.partial(_apply_body, cnt=rows * K),
        grid=(rows // tr,),
        in_specs=[
            pl.BlockSpec((tr, c1), lambda r: (r, 0)),
            pl.BlockSpec((2, c1), lambda r: (0, 0)),
            pl.BlockSpec((1, c1), lambda r: (0, 0)),
            pl.BlockSpec((1, c1), lambda r: (0, 0)),
        ],
        out_specs=pl.BlockSpec((tr, c1), lambda r: (r, 0)),
        out_shape=jax.ShapeDtypeStruct((rows, c1), jnp.float32),
    )(smax, acc, gamma, beta)


def _edge_block(featT, sourceCM, w, gamma, beta, tw):
    B, N, C = featT.shape
    c1 = w.shape[0]
    idx = _knn_call(featT, sourceCM)
    table = featT.reshape(B * N, C)
    if tw > C:
        table = jnp.pad(table, ((0, 0), (0, tw - C)))
    dpair = _sc_diff(table, idx.reshape(-1), C)
    smax, acc = _conv_stats(dpair, featT.reshape(B * N, C), w)
    out = _apply_call(smax, acc, gamma.reshape(1, c1), beta.reshape(1, c1))
    return out.reshape(B, N, c1)


def kernel(a, b, W1, gamma1, beta1, W2, gamma2, beta2, k):
    del k
    aT = jnp.transpose(a, (0, 2, 1))
    feat1 = _edge_block(aT, b, W1, gamma1, beta1, tw=16)
    feat1T = jnp.transpose(feat1, (0, 2, 1))
    feat2 = _edge_block(feat1, feat1T, W2, gamma2, beta2, tw=32)
    return feat2


# speedup vs baseline: 1.0087x; 1.0087x over previous
import functools

import jax
import jax.numpy as jnp
from jax import lax
from jax.experimental import pallas as pl
from jax.experimental.pallas import tpu as pltpu
from jax.experimental.pallas import tpu_sc as plsc

TQ = 512
QT = 256
K = 16
NW = 32
SC_CHUNK = 128


def _knn_body(aT_ref, s_ref, idx_ref, *, M):
    b = pl.program_id(0)
    q = aT_ref[0]
    s = s_ref[0]
    inner = lax.dot_general(q.astype(jnp.bfloat16), s.astype(jnp.bfloat16),
                            (((1,), (0,)), ((), ())),
                            preferred_element_type=jnp.float32)
    qq = jnp.sum(q * q, axis=1, keepdims=True)
    ss = jnp.sum(s * s, axis=0, keepdims=True)
    p = (2.0 * inner - qq) - ss

    iota = lax.broadcasted_iota(jnp.int32, (TQ, M), 1).astype(jnp.float32)
    bigf = jnp.float32(M)
    cols = []
    for t in range(K):
        m = jnp.max(p, axis=1, keepdims=True)
        cand = jnp.where(p == m, iota, bigf)
        sel = jnp.min(cand, axis=1, keepdims=True)
        cols.append(sel)
        if t < K - 1:
            p = jnp.where(iota == sel, -jnp.inf, p)
    idx = jnp.concatenate(cols, axis=1).astype(jnp.int32)
    idx_ref[0] = idx + b * jnp.int32(M)


def _knn_call(aT, s):
    B, N, C = aT.shape
    M = s.shape[2]
    nt = N // TQ
    return pl.pallas_call(
        functools.partial(_knn_body, M=M),
        grid=(B, nt),
        in_specs=[
            pl.BlockSpec((1, TQ, C), lambda b, t: (b, t, 0)),
            pl.BlockSpec((1, C, M), lambda b, t: (b, 0, 0)),
        ],
        out_specs=pl.BlockSpec((1, TQ, K), lambda b, t: (b, t, 0)),
        out_shape=jax.ShapeDtypeStruct((B, N, K), jnp.int32),
    )(aT, s)


def _srl(x, n):
    return lax.shift_right_logical(x, jnp.full(x.shape, n, x.dtype))


NBUF = 8


def _bf16_round(v):
    u = plsc.bitcast(v, jnp.int32)
    u = u + jnp.int32(0x7FFF) + (_srl(u, 16) & jnp.int32(1))
    u = u & jnp.int32(-65536)
    return plsc.bitcast(u, jnp.float32)


def _sc_diff_body(tab_hbm, idx_hbm, dout_hbm,
                  own_v, idx_v, rows_v, dbuf_v, gsem, osem, *, tw, c_real, qw):
    wid = lax.axis_index("s") * 2 + lax.axis_index("c")
    nchunks = (qw * K) // SC_CHUNK
    qpc = SC_CHUNK // K
    nv = tw // 16
    pltpu.sync_copy(tab_hbm.at[pl.ds(wid * qw, qw)], own_v)
    pltpu.sync_copy(idx_hbm.at[pl.ds(wid * nchunks, nchunks)], idx_v)

    for bi in range(NBUF):
        pltpu.async_copy(tab_hbm.at[idx_v.at[bi]], rows_v.at[bi], gsem)

    def group(g, carry):
        for bi in range(NBUF):
            cc = g * NBUF + bi
            pltpu.make_async_copy(tab_hbm.at[pl.ds(0, SC_CHUNK)],
                                  rows_v.at[bi], gsem).wait()
            @pl.when(cc >= NBUF)
            def _drain():
                pltpu.make_async_copy(
                    dbuf_v.at[bi],
                    dout_hbm.at[pl.ds(0, SC_CHUNK)], osem).wait()

            def qloop(q, inner_carry):
                qg = cc * qpc + q
                for v in range(nv):
                    fn = own_v[qg, pl.ds(v * 16, 16)]
                    for i in range(K):
                        r = q * K + i
                        fj = rows_v[bi, r, pl.ds(v * 16, 16)]
                        dbuf_v[bi, r, pl.ds(v * 16, 16)] = _bf16_round(fj - fn)
                return inner_carry

            lax.fori_loop(0, qpc, qloop, 0)
            pltpu.async_copy(
                dbuf_v.at[bi],
                dout_hbm.at[pl.ds(wid * qw * K + cc * SC_CHUNK, SC_CHUNK)],
                osem)
            @pl.when(cc + NBUF < nchunks)
            def _prefetch():
                pltpu.async_copy(tab_hbm.at[idx_v.at[cc + NBUF]],
                                 rows_v.at[bi], gsem)
        return carry

    lax.fori_loop(0, nchunks // NBUF, group, 0)
    for bi in range(NBUF):
        pltpu.make_async_copy(dbuf_v.at[bi], dout_hbm.at[pl.ds(0, SC_CHUNK)],
                              osem).wait()


def _sc_diff(table, idx_flat, c_real):
    rows, tw = table.shape
    nq = idx_flat.shape[0] // K
    qw = nq // NW
    idx2d = idx_flat.reshape(nq * K // SC_CHUNK, SC_CHUNK)
    mesh = plsc.VectorSubcoreMesh(core_axis_name="c", subcore_axis_name="s")
    fn = pl.kernel(
        functools.partial(_sc_diff_body, tw=tw, c_real=c_real, qw=qw),
        mesh=mesh,
        compiler_params=pltpu.CompilerParams(use_tc_tiling_on_sc=False,
                                             needs_layout_passes=False),
        out_type=jax.ShapeDtypeStruct((nq * K, tw), jnp.float32),
        scratch_types=[
            pltpu.VMEM((qw, tw), jnp.float32),
            pltpu.VMEM((qw * K // SC_CHUNK, SC_CHUNK), jnp.int32),
            pltpu.VMEM((NBUF, SC_CHUNK, tw), jnp.float32),
            pltpu.VMEM((NBUF, SC_CHUNK, tw), jnp.float32),
            pltpu.SemaphoreType.DMA,
            pltpu.SemaphoreType.DMA,
        ],
    )
    return fn(table, idx2d)


def _conv_stats_body(d_ref, f_ref, w_ref, smax_ref, acc_ref, *, C):
    d = d_ref[...]
    w = w_ref[...].astype(jnp.bfloat16)
    wl = w[:, :C]
    wr = w[:, C:]
    db = d[:, :C].astype(jnp.bfloat16)
    xd = lax.dot_general(db, wl, (((1,), (1,)), ((), ())),
                         preferred_element_type=jnp.float32)
    f = f_ref[...].astype(jnp.bfloat16)
    hc = lax.dot_general(f, wr, (((1,), (1,)), ((), ())),
                         preferred_element_type=jnp.float32)
    c1 = xd.shape[1]
    x3 = xd.reshape(QT, K, c1)
    smax0 = jnp.max(x3, axis=1)
    ssum0 = jnp.sum(x3, axis=1)
    ssq0 = jnp.sum(x3 * x3, axis=1)
    kf = jnp.float32(K)
    smax_ref[...] = smax0 + hc
    ssum = ssum0 + kf * hc
    ssq = ssq0 + 2.0 * hc * ssum0 + kf * hc * hc
    part = jnp.concatenate([jnp.sum(ssum, axis=0, keepdims=True),
                            jnp.sum(ssq, axis=0, keepdims=True)], axis=0)

    @pl.when(pl.program_id(0) == 0)
    def _init():
        acc_ref[...] = jnp.zeros_like(acc_ref)

    acc_ref[...] += part


def _conv_stats(dpair, feat, w):
    nqk, tw = dpair.shape
    nq = nqk // K
    C = feat.shape[1]
    c1 = w.shape[0]
    return pl.pallas_call(
        functools.partial(_conv_stats_body, C=C),
        grid=(nq // QT,),
        in_specs=[
            pl.BlockSpec((QT * K, tw), lambda r: (r, 0)),
            pl.BlockSpec((QT, C), lambda r: (r, 0)),
            pl.BlockSpec((c1, 2 * C), lambda r: (0, 0)),
        ],
        out_specs=[
            pl.BlockSpec((QT, c1), lambda r: (r, 0)),
            pl.BlockSpec((2, c1), lambda r: (0, 0)),
        ],
        out_shape=[
            jax.ShapeDtypeStruct((nq, c1), jnp.float32),
            jax.ShapeDtypeStruct((2, c1), jnp.float32),
        ],
    )(dpair, feat, w)


def _apply_body(smax_ref, acc_ref, gam_ref, bet_ref, out_ref, *, cnt):
    x = smax_ref[...]
    cntf = jnp.float32(cnt)
    mean = acc_ref[0:1, :] / cntf
    var = acc_ref[1:2, :] / cntf - mean * mean
    rstd = 1.0 / jnp.sqrt(var + 1e-5)
    y = (x - mean) * rstd * gam_ref[...] + bet_ref[...]
    out_ref[...] = jnp.where(y >= 0, y, 0.2 * y)


def _apply_call(smax, acc, gamma, beta):
    rows, c1 = smax.shape
    tr = min(2048, rows)
    return pl.pallas_call(
        functools.partial(_apply_body, cnt=rows * K),
        grid=(rows // tr,),
        in_specs=[
            pl.BlockSpec((tr, c1), lambda r: (r, 0)),
            pl.BlockSpec((2, c1), lambda r: (0, 0)),
            pl.BlockSpec((1, c1), lambda r: (0, 0)),
            pl.BlockSpec((1, c1), lambda r: (0, 0)),
        ],
        out_specs=pl.BlockSpec((tr, c1), lambda r: (r, 0)),
        out_shape=jax.ShapeDtypeStruct((rows, c1), jnp.float32),
    )(smax, acc, gamma, beta)


def _edge_block(featT, sourceCM, w, gamma, beta, tw):
    B, N, C = featT.shape
    c1 = w.shape[0]
    idx = _knn_call(featT, sourceCM)
    table = featT.reshape(B * N, C)
    if tw > C:
        table = jnp.pad(table, ((0, 0), (0, tw - C)))
    dpair = _sc_diff(table, idx.reshape(-1), C)
    smax, acc = _conv_stats(dpair, featT.reshape(B * N, C), w)
    out = _apply_call(smax, acc, gamma.reshape(1, c1), beta.reshape(1, c1))
    return out.reshape(B, N, c1)


def kernel(a, b, W1, gamma1, beta1, W2, gamma2, beta2, k):
    del k
    aT = jnp.transpose(a, (0, 2, 1))
    feat1 = _edge_block(aT, b, W1, gamma1, beta1, tw=16)
    feat1T = jnp.transpose(feat1, (0, 2, 1))
    feat2 = _edge_block(feat1, feat1T, W2, gamma2, beta2, tw=32)
    return feat2
